# Initial kernel scaffold; baseline (speedup 1.0000x reference)
import jax
import jax.numpy as jnp
from jax.experimental import pallas as pl


def kernel(x, pos_embedding, W_qkv, W_out, b_out):
    raise NotImplementedError("write your pallas kernel here")


# speedup vs baseline: 7.4780x; 7.4780x over previous
import functools

import jax
import jax.numpy as jnp
import numpy as np
from jax import lax
from jax.experimental import pallas as pl
from jax.experimental.pallas import tpu as pltpu
from jax.experimental.pallas import tpu_sc as plsc

B, T, N, DIM = 2, 4, 4096, 256
HEADS, DIM_HEAD, POOL = 8, 64, 8
M = N // POOL
KK = POOL + 1
INNER = HEADS * DIM_HEAD
RD = DIM_HEAD // 2
SCALE = DIM_HEAD ** -0.5
G = B * T * M
ROWS = KK * G
MB = 128

f32 = jnp.float32
bf16 = jnp.bfloat16


def _fps_body(px_ref, py_ref, pivx_ref, pivy_ref):
    px = px_ref[...]
    py = py_ref[...]
    lane = lax.broadcasted_iota(jnp.int32, (B, N), 1)
    col = lax.broadcasted_iota(jnp.int32, (B, M), 1)
    x0 = px[:, 0:1]
    y0 = py[:, 0:1]
    dx = px - x0
    dy = py - y0
    mind0 = dx * dx + dy * dy
    pivx_ref[...] = jnp.where(col == 0, x0, 0.0)
    pivy_ref[...] = jnp.where(col == 0, y0, 0.0)

    def step(i, mind):
        amax = jnp.max(mind, axis=1, keepdims=True)
        idx = jnp.min(jnp.where(mind == amax, lane, N), axis=1, keepdims=True)
        sel = lane == idx
        nx = jnp.sum(jnp.where(sel, px, 0.0), axis=1, keepdims=True)
        ny = jnp.sum(jnp.where(sel, py, 0.0), axis=1, keepdims=True)
        ddx = px - nx
        ddy = py - ny
        nd = ddx * ddx + ddy * ddy
        pivx_ref[...] = jnp.where(col == i, nx, pivx_ref[...])
        pivy_ref[...] = jnp.where(col == i, ny, pivy_ref[...])
        return jnp.minimum(mind, nd)

    lax.fori_loop(1, M, step, mind0)


def _fps(px, py):
    return pl.pallas_call(
        _fps_body,
        out_shape=[jax.ShapeDtypeStruct((B, M), f32),
                   jax.ShapeDtypeStruct((B, M), f32)],
    )(px, py)


def _knn_body(pxc_ref, pyc_ref, pivx_ref, pivy_ref,
              nbr_ref, relx_ref, rely_ref, pd_ref):
    pxc = pxc_ref[0]
    pyc = pyc_ref[0]
    pivx = pivx_ref[0]
    pivy = pivy_ref[0]
    pxb = pxc.astype(bf16).astype(f32)
    pyb = pyc.astype(bf16).astype(f32)
    pvxb = pivx.astype(bf16).astype(f32)
    pvyb = pivy.astype(bf16).astype(f32)
    t = pvxb * pxb + pvyb * pyb
    inner = -2.0 * t
    xx = pivx * pivx + pivy * pivy
    yy = pxc * pxc + pyc * pyc
    pd_ref[...] = (-xx) - inner - yy
    rows = lax.broadcasted_iota(jnp.int32, (N, M), 0)
    for j in range(KK):
        pdv = pd_ref[...]
        amax = jnp.max(pdv, axis=0, keepdims=True)
        idx = jnp.min(jnp.where(pdv == amax, rows, N), axis=0, keepdims=True)
        sel = rows == idx
        gx = jnp.sum(jnp.where(sel, pxc, 0.0), axis=0, keepdims=True)
        gy = jnp.sum(jnp.where(sel, pyc, 0.0), axis=0, keepdims=True)
        nbr_ref[0, j] = idx[0]
        relx_ref[0, j] = (gx - pivx)[0]
        rely_ref[0, j] = (gy - pivy)[0]
        pd_ref[...] = jnp.where(sel, -jnp.inf, pdv)


def _knn(pxc, pyc, pivx, pivy):
    return pl.pallas_call(
        _knn_body,
        grid=(B,),
        in_specs=[
            pl.BlockSpec((1, N, 1), lambda b: (b, 0, 0)),
            pl.BlockSpec((1, N, 1), lambda b: (b, 0, 0)),
            pl.BlockSpec((1, 1, M), lambda b: (b, 0, 0)),
            pl.BlockSpec((1, 1, M), lambda b: (b, 0, 0)),
        ],
        out_specs=[
            pl.BlockSpec((1, KK, M), lambda b: (b, 0, 0)),
            pl.BlockSpec((1, KK, M), lambda b: (b, 0, 0)),
            pl.BlockSpec((1, KK, M), lambda b: (b, 0, 0)),
        ],
        out_shape=[jax.ShapeDtypeStruct((B, KK, M), jnp.int32),
                   jax.ShapeDtypeStruct((B, KK, M), f32),
                   jax.ShapeDtypeStruct((B, KK, M), f32)],
        scratch_shapes=[pltpu.VMEM((N, M), f32)],
    )(pxc, pyc, pivx.reshape(B, 1, M), pivy.reshape(B, 1, M))


_NWRK = 32
_ROWS_W = ROWS // _NWRK
_CH = 128
_NCH = _ROWS_W // _CH


def _gather_body(xf_hbm, idx_hbm, out_hbm, idx_v, rows_a, rows_b, sem_a, sem_b):
    c = lax.axis_index("c")
    s = lax.axis_index("s")
    wid = s * 2 + c
    base = wid * _ROWS_W
    pltpu.sync_copy(idx_hbm.at[pl.ds(base, _ROWS_W)], idx_v)
    bufs = (rows_a, rows_b)
    sems = (sem_a, sem_b)
    cps = [None, None]
    cps[0] = pltpu.async_copy(xf_hbm.at[idx_v.at[pl.ds(0, _CH)]], rows_a, sem_a)
    for ch in range(_NCH):
        nxt = ch + 1
        if nxt < _NCH:
            cps[nxt % 2] = pltpu.async_copy(
                xf_hbm.at[idx_v.at[pl.ds(nxt * _CH, _CH)]],
                bufs[nxt % 2], sems[nxt % 2])
        cps[ch % 2].wait()
        pltpu.sync_copy(bufs[ch % 2], out_hbm.at[pl.ds(base + ch * _CH, _CH)])


def _gather(xf, flat_idx):
    mesh = plsc.VectorSubcoreMesh(core_axis_name="c", subcore_axis_name="s")
    k = functools.partial(
        pl.kernel,
        mesh=mesh,
        out_type=jax.ShapeDtypeStruct((ROWS, DIM), f32),
        scratch_types=[
            pltpu.VMEM((_ROWS_W,), jnp.int32),
            pltpu.VMEM((_CH, DIM), f32),
            pltpu.VMEM((_CH, DIM), f32),
            pltpu.SemaphoreType.DMA,
            pltpu.SemaphoreType.DMA,
        ],
    )(_gather_body)
    return k(xf, flat_idx)


def _dense_body(xg_ref, rxT_ref, ryT_ref, wc_ref, w2_ref, axay_ref,
                sbig_ref, tmat_ref, tmatT_ref, ub_ref, wout_ref, bout_ref,
                out_ref):
    rxT = rxT_ref[0]
    ryT = ryT_ref[0]
    ax = axay_ref[0:1, :]
    ay = axay_ref[1:2, :]
    w2q = w2_ref[0:1, :]
    w2r = w2_ref[1:2, :]
    qs, ks, vs = [], [], []
    for k in range(KK):
        rxc = rxT[:, k:k + 1]
        ryc = ryT[:, k:k + 1]
        xk = xg_ref[k].astype(bf16)
        qkv = jnp.dot(xk, wc_ref[...], preferred_element_type=f32)
        qkv = qkv + rxc * w2q + ryc * w2r
        f64 = rxc * ax + ryc * ay
        c64 = jnp.cos(f64)
        s64 = jnp.sin(f64)
        cT = jnp.concatenate([c64] * HEADS, axis=1)
        sT = jnp.concatenate([s64] * HEADS, axis=1)
        q = qkv[:, 0:INNER]
        qP = qkv[:, INNER:2 * INNER]
        kq = qkv[:, 2 * INNER:3 * INNER]
        kP = qkv[:, 3 * INNER:4 * INNER]
        v = qkv[:, 4 * INNER:5 * INNER]
        qr = (q * cT + qP * sT) * SCALE
        kr = kq * cT + kP * sT
        qs.append(qr.astype(bf16))
        ks.append(kr.astype(bf16))
        vs.append(v)
    aacc = jnp.zeros((MB, HEADS * KK), dtype=f32)
    for i in range(KK):
        zi = jnp.concatenate([qs[i] * ks[j] for j in range(KK)], axis=1)
        di = jnp.dot(zi, sbig_ref[...], preferred_element_type=f32)
        ei = jnp.exp(di)
        si = jnp.dot(ei.astype(bf16), tmat_ref[...], preferred_element_type=f32)
        ri = 1.0 / si
        rexp = jnp.dot(ri.astype(bf16), tmatT_ref[...],
                       preferred_element_type=f32)
        aacc = aacc + ei * rexp
    abar = (aacc * (1.0 / KK)).astype(bf16)
    pooled = jnp.zeros((MB, INNER), dtype=f32)
    for j in range(KK):
        aexp = jnp.dot(abar, ub_ref[j], preferred_element_type=f32)
        pooled = pooled + aexp * vs[j]
    out = jnp.dot(pooled.astype(bf16), wout_ref[...], preferred_element_type=f32)
    out_ref[0, 0] = out + bout_ref[...]


def _dense(xg, rxT, ryT, wc, w2, axay, sbig, tmat, tmatT, ub, wout, bout):
    nj = M // MB
    return pl.pallas_call(
        _dense_body,
        grid=(B, T, nj),
        in_specs=[
            pl.BlockSpec((KK, MB, DIM),
                         lambda b, t, j: (0, (b * T + t) * nj + j, 0)),
            pl.BlockSpec((1, MB, KK), lambda b, t, j: (b, j, 0)),
            pl.BlockSpec((1, MB, KK), lambda b, t, j: (b, j, 0)),
            pl.BlockSpec((DIM, 5 * INNER), lambda b, t, j: (0, 0)),
            pl.BlockSpec((2, 5 * INNER), lambda b, t, j: (0, 0)),
            pl.BlockSpec((2, DIM_HEAD), lambda b, t, j: (0, 0)),
            pl.BlockSpec((KK * INNER, HEADS * KK), lambda b, t, j: (0, 0)),
            pl.BlockSpec((HEADS * KK, HEADS), lambda b, t, j: (0, 0)),
            pl.BlockSpec((HEADS, HEADS * KK), lambda b, t, j: (0, 0)),
            pl.BlockSpec((KK, HEADS * KK, INNER), lambda b, t, j: (0, 0, 0)),
            pl.BlockSpec((INNER, DIM), lambda b, t, j: (0, 0)),
            pl.BlockSpec((1, DIM), lambda b, t, j: (0, 0)),
        ],
        out_specs=pl.BlockSpec((1, 1, MB, DIM), lambda b, t, j: (b, t, j, 0)),
        out_shape=jax.ShapeDtypeStruct((B, T, M, DIM), f32),
    )(xg.reshape(KK, G, DIM), rxT, ryT, wc, w2, axay, sbig, tmat, tmatT,
      ub, wout, bout)


def _structure_mats():
    sbig = np.zeros((KK * INNER, HEADS * KK), dtype=np.float32)
    for j in range(KK):
        for h in range(HEADS):
            sbig[j * INNER + h * DIM_HEAD:(j * INNER + (h + 1) * DIM_HEAD),
                 h * KK + j] = 1.0
    tmat = np.zeros((HEADS * KK, HEADS), dtype=np.float32)
    for h in range(HEADS):
        tmat[h * KK:(h + 1) * KK, h] = 1.0
    ub = np.zeros((KK, HEADS * KK, INNER), dtype=np.float32)
    for j in range(KK):
        for h in range(HEADS):
            ub[j, h * KK + j, h * DIM_HEAD:(h + 1) * DIM_HEAD] = 1.0
    return sbig, tmat, tmat.T, ub


_SBIG_NP, _TMAT_NP, _TMATT_NP, _UB_NP = _structure_mats()


def _rot_cols(w):
    wh = w.reshape(-1, HEADS, 4, RD // 2)
    a, b2, c, d = wh[:, :, 0], wh[:, :, 1], wh[:, :, 2], wh[:, :, 3]
    return jnp.stack([-b2, a, -d, c], axis=2).reshape(-1, INNER)


def kernel(x, pos_embedding, W_qkv, W_out, b_out):
    px = pos_embedding[:, :, 0]
    py = pos_embedding[:, :, 1]
    pivx, pivy = _fps(px, py)
    nbr, relx, rely = _knn(px[..., None], py[..., None], pivx, pivy)

    bt_base = (jnp.arange(B * T, dtype=jnp.int32) * N).reshape(B, 1, T, 1)
    flat = (nbr[:, :, None, :] + bt_base)
    flat = jnp.transpose(flat, (1, 0, 2, 3)).reshape(ROWS)
    xg = _gather(x.reshape(B * T * N, DIM), flat)

    wq = W_qkv[:, 0:INNER]
    wk = W_qkv[:, INNER:2 * INNER]
    wv = W_qkv[:, 2 * INNER:3 * INNER]
    wcat = jnp.concatenate(
        [wq, _rot_cols(wq), wk, _rot_cols(wk), wv], axis=1)
    wc = wcat[:DIM].astype(bf16)
    w2 = wcat[DIM:DIM + 2]
    invf = 1.0 / (10000.0 ** (jnp.arange(0, RD, 2, dtype=f32) / RD))
    base = jnp.concatenate([invf, invf]) * 2048.0
    zeros = jnp.zeros((RD,), dtype=f32)
    axay = jnp.stack([jnp.concatenate([base, zeros]),
                      jnp.concatenate([zeros, base])])

    out = _dense(xg, jnp.swapaxes(relx, 1, 2), jnp.swapaxes(rely, 1, 2),
                 wc, w2, axay,
                 jnp.asarray(_SBIG_NP, dtype=bf16),
                 jnp.asarray(_TMAT_NP, dtype=bf16),
                 jnp.asarray(_TMATT_NP, dtype=bf16),
                 jnp.asarray(_UB_NP, dtype=bf16),
                 W_out.astype(bf16), b_out.reshape(1, DIM))
    pivot_pos = jnp.stack([pivx, pivy], axis=-1)
    return out, pivot_pos


# speedup vs baseline: 7.8049x; 1.0437x over previous
import functools

import jax
import jax.numpy as jnp
import numpy as np
from jax import lax
from jax.experimental import pallas as pl
from jax.experimental.pallas import tpu as pltpu
from jax.experimental.pallas import tpu_sc as plsc

B, T, N, DIM = 2, 4, 4096, 256
HEADS, DIM_HEAD, POOL = 8, 64, 8
M = N // POOL
KK = POOL + 1
INNER = HEADS * DIM_HEAD
RD = DIM_HEAD // 2
SCALE = DIM_HEAD ** -0.5
G = B * T * M
ROWS = KK * G
MB = 128

f32 = jnp.float32
bf16 = jnp.bfloat16


_FR = N // 128


def _fps_body(px_ref, py_ref, pivx_ref, pivy_ref):
    px = px_ref[...].reshape(B, _FR, 128)
    py = py_ref[...].reshape(B, _FR, 128)
    r32 = lax.broadcasted_iota(jnp.int32, (B, _FR, 128), 1)
    lane = lax.broadcasted_iota(jnp.int32, (B, _FR, 128), 2)
    gid = r32 * 128 + lane
    col = lax.broadcasted_iota(jnp.int32, (B, M), 1)
    x0 = px[:, 0:1, 0:1]
    y0 = py[:, 0:1, 0:1]
    dx = px - x0
    dy = py - y0
    mind0 = dx * dx + dy * dy
    pivx_ref[...] = jnp.where(col == 0, x0[:, 0], 0.0)
    pivy_ref[...] = jnp.where(col == 0, y0[:, 0], 0.0)

    def step(i, mind):
        am = jnp.max(jnp.max(mind, axis=1), axis=1)[:, None, None]
        idx = jnp.where(mind == am, gid, N)
        idx = jnp.min(jnp.min(idx, axis=1), axis=1)[:, None, None]
        sel = gid == idx
        nx = jnp.sum(jnp.sum(jnp.where(sel, px, 0.0), axis=1), axis=1)
        ny = jnp.sum(jnp.sum(jnp.where(sel, py, 0.0), axis=1), axis=1)
        nxb = nx[:, None, None]
        nyb = ny[:, None, None]
        ddx = px - nxb
        ddy = py - nyb
        nd = ddx * ddx + ddy * ddy
        pivx_ref[...] = jnp.where(col == i, nx[:, None], pivx_ref[...])
        pivy_ref[...] = jnp.where(col == i, ny[:, None], pivy_ref[...])
        return jnp.minimum(mind, nd)

    lax.fori_loop(1, M, step, mind0)


def _fps(px, py):
    return pl.pallas_call(
        _fps_body,
        out_shape=[jax.ShapeDtypeStruct((B, M), f32),
                   jax.ShapeDtypeStruct((B, M), f32)],
    )(px.reshape(B * _FR, 128), py.reshape(B * _FR, 128))


def _knn_body(pxc_ref, pyc_ref, pivx_ref, pivy_ref,
              nbr_ref, relx_ref, rely_ref, pd_ref):
    pxc = pxc_ref[0]
    pyc = pyc_ref[0]
    pivx = pivx_ref[0]
    pivy = pivy_ref[0]
    pxb = pxc.astype(bf16).astype(f32)
    pyb = pyc.astype(bf16).astype(f32)
    pvxb = pivx.astype(bf16).astype(f32)
    pvyb = pivy.astype(bf16).astype(f32)
    t = pvxb * pxb + pvyb * pyb
    inner = -2.0 * t
    xx = pivx * pivx + pivy * pivy
    yy = pxc * pxc + pyc * pyc
    pd_ref[...] = (-xx) - inner - yy
    rows = lax.broadcasted_iota(jnp.int32, (N, M), 0)
    for j in range(KK):
        pdv = pd_ref[...]
        amax = jnp.max(pdv, axis=0, keepdims=True)
        idx = jnp.min(jnp.where(pdv == amax, rows, N), axis=0, keepdims=True)
        sel = rows == idx
        gx = jnp.sum(jnp.where(sel, pxc, 0.0), axis=0, keepdims=True)
        gy = jnp.sum(jnp.where(sel, pyc, 0.0), axis=0, keepdims=True)
        nbr_ref[0, j] = idx[0]
        relx_ref[0, j] = (gx - pivx)[0]
        rely_ref[0, j] = (gy - pivy)[0]
        pd_ref[...] = jnp.where(sel, -jnp.inf, pdv)


def _knn(pxc, pyc, pivx, pivy):
    return pl.pallas_call(
        _knn_body,
        grid=(B,),
        in_specs=[
            pl.BlockSpec((1, N, 1), lambda b: (b, 0, 0)),
            pl.BlockSpec((1, N, 1), lambda b: (b, 0, 0)),
            pl.BlockSpec((1, 1, M), lambda b: (b, 0, 0)),
            pl.BlockSpec((1, 1, M), lambda b: (b, 0, 0)),
        ],
        out_specs=[
            pl.BlockSpec((1, KK, M), lambda b: (b, 0, 0)),
            pl.BlockSpec((1, KK, M), lambda b: (b, 0, 0)),
            pl.BlockSpec((1, KK, M), lambda b: (b, 0, 0)),
        ],
        out_shape=[jax.ShapeDtypeStruct((B, KK, M), jnp.int32),
                   jax.ShapeDtypeStruct((B, KK, M), f32),
                   jax.ShapeDtypeStruct((B, KK, M), f32)],
        scratch_shapes=[pltpu.VMEM((N, M), f32)],
    )(pxc, pyc, pivx.reshape(B, 1, M), pivy.reshape(B, 1, M))


_NWRK = 32
_ROWS_W = ROWS // _NWRK
_CH = 128
_NCH = _ROWS_W // _CH


def _gather_body(xf_hbm, idx_hbm, out_hbm, idx_v, rows_a, rows_b, sem_a, sem_b):
    c = lax.axis_index("c")
    s = lax.axis_index("s")
    wid = s * 2 + c
    base = wid * _ROWS_W
    pltpu.sync_copy(idx_hbm.at[pl.ds(base, _ROWS_W)], idx_v)
    bufs = (rows_a, rows_b)
    sems = (sem_a, sem_b)
    cps = [None, None]
    cps[0] = pltpu.async_copy(xf_hbm.at[idx_v.at[pl.ds(0, _CH)]], rows_a, sem_a)
    for ch in range(_NCH):
        nxt = ch + 1
        if nxt < _NCH:
            cps[nxt % 2] = pltpu.async_copy(
                xf_hbm.at[idx_v.at[pl.ds(nxt * _CH, _CH)]],
                bufs[nxt % 2], sems[nxt % 2])
        cps[ch % 2].wait()
        pltpu.sync_copy(bufs[ch % 2], out_hbm.at[pl.ds(base + ch * _CH, _CH)])


def _gather(xf, flat_idx):
    mesh = plsc.VectorSubcoreMesh(core_axis_name="c", subcore_axis_name="s")
    k = functools.partial(
        pl.kernel,
        mesh=mesh,
        out_type=jax.ShapeDtypeStruct((ROWS, DIM), f32),
        scratch_types=[
            pltpu.VMEM((_ROWS_W,), jnp.int32),
            pltpu.VMEM((_CH, DIM), f32),
            pltpu.VMEM((_CH, DIM), f32),
            pltpu.SemaphoreType.DMA,
            pltpu.SemaphoreType.DMA,
        ],
    )(_gather_body)
    return k(xf, flat_idx)


def _dense_body(xg_ref, rxc_ref, ryc_ref, wc_ref, w2_ref, axay_ref,
                sbig_ref, tmat_ref, tmatT_ref, ub_ref, wout_ref, bout_ref,
                out_ref):
    rxc = rxc_ref[0].reshape(KK * MB, 1)
    ryc = ryc_ref[0].reshape(KK * MB, 1)
    ax = axay_ref[0:1, :]
    ay = axay_ref[1:2, :]
    w2q = w2_ref[0:1, :]
    w2r = w2_ref[1:2, :]
    xall = xg_ref[...].reshape(KK * MB, DIM).astype(bf16)
    qkv = jnp.dot(xall, wc_ref[...], preferred_element_type=f32)
    qkv = qkv + rxc * w2q + ryc * w2r
    f64 = rxc * ax + ryc * ay
    c64 = jnp.cos(f64)
    s64 = jnp.sin(f64)
    cT = jnp.concatenate([c64] * HEADS, axis=1)
    sT = jnp.concatenate([s64] * HEADS, axis=1)
    q = qkv[:, 0:INNER]
    qP = qkv[:, INNER:2 * INNER]
    kq = qkv[:, 2 * INNER:3 * INNER]
    kP = qkv[:, 3 * INNER:4 * INNER]
    qr = ((q * cT + qP * sT) * SCALE).astype(bf16)
    kr = (kq * cT + kP * sT).astype(bf16)
    krcat = jnp.concatenate(
        [kr[j * MB:(j + 1) * MB] for j in range(KK)], axis=1)
    aacc = jnp.zeros((MB, HEADS * KK), dtype=f32)
    es = []
    for i in range(KK):
        qi = qr[i * MB:(i + 1) * MB]
        zi = jnp.concatenate([qi] * KK, axis=1) * krcat
        di = jnp.dot(zi, sbig_ref[...], preferred_element_type=f32)
        ei = jnp.exp(di)
        si = jnp.dot(ei.astype(bf16), tmat_ref[...], preferred_element_type=f32)
        ri = 1.0 / si
        rexp = jnp.dot(ri.astype(bf16), tmatT_ref[...],
                       preferred_element_type=f32)
        aacc = aacc + ei * rexp
    abar = (aacc * (1.0 / KK)).astype(bf16)
    pooled = jnp.zeros((MB, INNER), dtype=f32)
    for j in range(KK):
        aexp = jnp.dot(abar, ub_ref[j], preferred_element_type=f32)
        pooled = pooled + aexp * qkv[:, 4 * INNER:5 * INNER][j * MB:(j + 1) * MB]
    out = jnp.dot(pooled.astype(bf16), wout_ref[...], preferred_element_type=f32)
    out_ref[0, 0] = out + bout_ref[...]


def _dense(xg, rxc, ryc, wc, w2, axay, sbig, tmat, tmatT, ub, wout, bout):
    nj = M // MB
    return pl.pallas_call(
        _dense_body,
        grid=(B, T, nj),
        in_specs=[
            pl.BlockSpec((KK, MB, DIM),
                         lambda b, t, j: (0, (b * T + t) * nj + j, 0)),
            pl.BlockSpec((1, KK, MB, 1), lambda b, t, j: (b, 0, j, 0)),
            pl.BlockSpec((1, KK, MB, 1), lambda b, t, j: (b, 0, j, 0)),
            pl.BlockSpec((DIM, 5 * INNER), lambda b, t, j: (0, 0)),
            pl.BlockSpec((2, 5 * INNER), lambda b, t, j: (0, 0)),
            pl.BlockSpec((2, DIM_HEAD), lambda b, t, j: (0, 0)),
            pl.BlockSpec((KK * INNER, HEADS * KK), lambda b, t, j: (0, 0)),
            pl.BlockSpec((HEADS * KK, HEADS), lambda b, t, j: (0, 0)),
            pl.BlockSpec((HEADS, HEADS * KK), lambda b, t, j: (0, 0)),
            pl.BlockSpec((KK, HEADS * KK, INNER), lambda b, t, j: (0, 0, 0)),
            pl.BlockSpec((INNER, DIM), lambda b, t, j: (0, 0)),
            pl.BlockSpec((1, DIM), lambda b, t, j: (0, 0)),
        ],
        out_specs=pl.BlockSpec((1, 1, MB, DIM), lambda b, t, j: (b, t, j, 0)),
        out_shape=jax.ShapeDtypeStruct((B, T, M, DIM), f32),
    )(xg.reshape(KK, G, DIM), rxc, ryc, wc, w2, axay, sbig, tmat, tmatT,
      ub, wout, bout)


def _structure_mats():
    sbig = np.zeros((KK * INNER, HEADS * KK), dtype=np.float32)
    for j in range(KK):
        for h in range(HEADS):
            sbig[j * INNER + h * DIM_HEAD:(j * INNER + (h + 1) * DIM_HEAD),
                 h * KK + j] = 1.0
    tmat = np.zeros((HEADS * KK, HEADS), dtype=np.float32)
    for h in range(HEADS):
        tmat[h * KK:(h + 1) * KK, h] = 1.0
    ub = np.zeros((KK, HEADS * KK, INNER), dtype=np.float32)
    for j in range(KK):
        for h in range(HEADS):
            ub[j, h * KK + j, h * DIM_HEAD:(h + 1) * DIM_HEAD] = 1.0
    return sbig, tmat, tmat.T, ub


_SBIG_NP, _TMAT_NP, _TMATT_NP, _UB_NP = _structure_mats()


def _rot_cols(w):
    wh = w.reshape(-1, HEADS, 4, RD // 2)
    a, b2, c, d = wh[:, :, 0], wh[:, :, 1], wh[:, :, 2], wh[:, :, 3]
    return jnp.stack([-b2, a, -d, c], axis=2).reshape(-1, INNER)


def kernel(x, pos_embedding, W_qkv, W_out, b_out):
    px = pos_embedding[:, :, 0]
    py = pos_embedding[:, :, 1]
    pivx, pivy = _fps(px, py)
    nbr, relx, rely = _knn(px[..., None], py[..., None], pivx, pivy)

    bt_base = (jnp.arange(B * T, dtype=jnp.int32) * N).reshape(B, 1, T, 1)
    flat = (nbr[:, :, None, :] + bt_base)
    flat = jnp.transpose(flat, (1, 0, 2, 3)).reshape(ROWS)
    xg = _gather(x.reshape(B * T * N, DIM), flat)

    wq = W_qkv[:, 0:INNER]
    wk = W_qkv[:, INNER:2 * INNER]
    wv = W_qkv[:, 2 * INNER:3 * INNER]
    wcat = jnp.concatenate(
        [wq, _rot_cols(wq), wk, _rot_cols(wk), wv], axis=1)
    wc = wcat[:DIM].astype(bf16)
    w2 = wcat[DIM:DIM + 2]
    invf = 1.0 / (10000.0 ** (jnp.arange(0, RD, 2, dtype=f32) / RD))
    base = jnp.concatenate([invf, invf]) * 2048.0
    zeros = jnp.zeros((RD,), dtype=f32)
    axay = jnp.stack([jnp.concatenate([base, zeros]),
                      jnp.concatenate([zeros, base])])

    out = _dense(xg, relx[..., None], rely[..., None],
                 wc, w2, axay,
                 jnp.asarray(_SBIG_NP, dtype=bf16),
                 jnp.asarray(_TMAT_NP, dtype=bf16),
                 jnp.asarray(_TMATT_NP, dtype=bf16),
                 jnp.asarray(_UB_NP, dtype=bf16),
                 W_out.astype(bf16), b_out.reshape(1, DIM))
    pivot_pos = jnp.stack([pivx, pivy], axis=-1)
    return out, pivot_pos


# speedup vs baseline: 8.5920x; 1.1008x over previous
import functools

import jax
import jax.numpy as jnp
import numpy as np
from jax import lax
from jax.experimental import pallas as pl
from jax.experimental.pallas import tpu as pltpu
from jax.experimental.pallas import tpu_sc as plsc

B, T, N, DIM = 2, 4, 4096, 256
HEADS, DIM_HEAD, POOL = 8, 64, 8
M = N // POOL
KK = POOL + 1
INNER = HEADS * DIM_HEAD
RD = DIM_HEAD // 2
SCALE = DIM_HEAD ** -0.5
G = B * T * M
ROWS = KK * G
MB = 128

f32 = jnp.float32
bf16 = jnp.bfloat16


_FR = N // 128


def _fps_body(px_ref, py_ref, pivx_ref, pivy_ref):
    px = px_ref[...].reshape(B, _FR, 128)
    py = py_ref[...].reshape(B, _FR, 128)
    r32 = lax.broadcasted_iota(jnp.int32, (B, _FR, 128), 1)
    lane = lax.broadcasted_iota(jnp.int32, (B, _FR, 128), 2)
    gid = r32 * 128 + lane
    col = lax.broadcasted_iota(jnp.int32, (B, M), 1)
    x0 = px[:, 0:1, 0:1]
    y0 = py[:, 0:1, 0:1]
    dx = px - x0
    dy = py - y0
    mind0 = dx * dx + dy * dy
    pivx_ref[...] = jnp.where(col == 0, x0[:, 0], 0.0)
    pivy_ref[...] = jnp.where(col == 0, y0[:, 0], 0.0)

    def step(i, mind):
        am = jnp.max(jnp.max(mind, axis=1), axis=1)[:, None, None]
        idx = jnp.where(mind == am, gid, N)
        idx = jnp.min(jnp.min(idx, axis=1), axis=1)[:, None, None]
        sel = gid == idx
        nx = jnp.sum(jnp.sum(jnp.where(sel, px, 0.0), axis=1), axis=1)
        ny = jnp.sum(jnp.sum(jnp.where(sel, py, 0.0), axis=1), axis=1)
        nxb = nx[:, None, None]
        nyb = ny[:, None, None]
        ddx = px - nxb
        ddy = py - nyb
        nd = ddx * ddx + ddy * ddy
        pivx_ref[...] = jnp.where(col == i, nx[:, None], pivx_ref[...])
        pivy_ref[...] = jnp.where(col == i, ny[:, None], pivy_ref[...])
        return jnp.minimum(mind, nd)

    lax.fori_loop(1, M, step, mind0)


def _fps(px, py):
    return pl.pallas_call(
        _fps_body,
        out_shape=[jax.ShapeDtypeStruct((B, M), f32),
                   jax.ShapeDtypeStruct((B, M), f32)],
    )(px.reshape(B * _FR, 128), py.reshape(B * _FR, 128))


def _knn_body(pxc_ref, pyc_ref, pivx_ref, pivy_ref,
              nbr_ref, relx_ref, rely_ref, pd_ref):
    pxc = pxc_ref[0]
    pyc = pyc_ref[0]
    pivx = pivx_ref[0]
    pivy = pivy_ref[0]
    pxb = pxc.astype(bf16).astype(f32)
    pyb = pyc.astype(bf16).astype(f32)
    pvxb = pivx.astype(bf16).astype(f32)
    pvyb = pivy.astype(bf16).astype(f32)
    t = pvxb * pxb + pvyb * pyb
    inner = -2.0 * t
    xx = pivx * pivx + pivy * pivy
    yy = pxc * pxc + pyc * pyc
    pd_ref[...] = (-xx) - inner - yy
    rows = lax.broadcasted_iota(jnp.int32, (N, M), 0)
    for j in range(KK):
        pdv = pd_ref[...]
        amax = jnp.max(pdv, axis=0, keepdims=True)
        idx = jnp.min(jnp.where(pdv == amax, rows, N), axis=0, keepdims=True)
        sel = rows == idx
        gx = jnp.sum(jnp.where(sel, pxc, 0.0), axis=0, keepdims=True)
        gy = jnp.sum(jnp.where(sel, pyc, 0.0), axis=0, keepdims=True)
        nbr_ref[0, j] = idx[0]
        relx_ref[0, j] = (gx - pivx)[0]
        rely_ref[0, j] = (gy - pivy)[0]
        pd_ref[...] = jnp.where(sel, -jnp.inf, pdv)


def _knn(pxc, pyc, pivx, pivy):
    return pl.pallas_call(
        _knn_body,
        grid=(B,),
        in_specs=[
            pl.BlockSpec((1, N, 1), lambda b: (b, 0, 0)),
            pl.BlockSpec((1, N, 1), lambda b: (b, 0, 0)),
            pl.BlockSpec((1, 1, M), lambda b: (b, 0, 0)),
            pl.BlockSpec((1, 1, M), lambda b: (b, 0, 0)),
        ],
        out_specs=[
            pl.BlockSpec((1, KK, M), lambda b: (b, 0, 0)),
            pl.BlockSpec((1, KK, M), lambda b: (b, 0, 0)),
            pl.BlockSpec((1, KK, M), lambda b: (b, 0, 0)),
        ],
        out_shape=[jax.ShapeDtypeStruct((B, KK, M), jnp.int32),
                   jax.ShapeDtypeStruct((B, KK, M), f32),
                   jax.ShapeDtypeStruct((B, KK, M), f32)],
        scratch_shapes=[pltpu.VMEM((N, M), f32)],
    )(pxc, pyc, pivx.reshape(B, 1, M), pivy.reshape(B, 1, M))


_NWRK = 32
_ROWS_W = ROWS // _NWRK
_CH = 128
_NCH = _ROWS_W // _CH


def _gather_body(xf_hbm, idx_hbm, out_hbm, idx_v, rows_a, rows_b, sem_a, sem_b):
    c = lax.axis_index("c")
    s = lax.axis_index("s")
    wid = s * 2 + c
    base = wid * _ROWS_W
    pltpu.sync_copy(idx_hbm.at[pl.ds(base, _ROWS_W)], idx_v)
    bufs = (rows_a, rows_b)
    sems = (sem_a, sem_b)
    cps = [None, None]
    cps[0] = pltpu.async_copy(xf_hbm.at[idx_v.at[pl.ds(0, _CH)]], rows_a, sem_a)
    for ch in range(_NCH):
        nxt = ch + 1
        if nxt < _NCH:
            cps[nxt % 2] = pltpu.async_copy(
                xf_hbm.at[idx_v.at[pl.ds(nxt * _CH, _CH)]],
                bufs[nxt % 2], sems[nxt % 2])
        cps[ch % 2].wait()
        pltpu.sync_copy(bufs[ch % 2], out_hbm.at[pl.ds(base + ch * _CH, _CH)])


def _gather(xf, flat_idx):
    mesh = plsc.VectorSubcoreMesh(core_axis_name="c", subcore_axis_name="s")
    k = functools.partial(
        pl.kernel,
        mesh=mesh,
        out_type=jax.ShapeDtypeStruct((ROWS, DIM), f32),
        scratch_types=[
            pltpu.VMEM((_ROWS_W,), jnp.int32),
            pltpu.VMEM((_CH, DIM), f32),
            pltpu.VMEM((_CH, DIM), f32),
            pltpu.SemaphoreType.DMA,
            pltpu.SemaphoreType.DMA,
        ],
    )(_gather_body)
    return k(xf, flat_idx)


def _dense_body(xg_ref, rxc_ref, ryc_ref, wc_ref, w2_ref, axay_ref,
                sbig_ref, tmat_ref, tmatT_ref, ub_ref, wout_ref, bout_ref,
                out_ref):
    rxc = rxc_ref[0].reshape(KK * MB, 1)
    ryc = ryc_ref[0].reshape(KK * MB, 1)
    ax = axay_ref[0:1, :]
    ay = axay_ref[1:2, :]
    w2q = w2_ref[0:1, :]
    w2r = w2_ref[1:2, :]
    xall = xg_ref[...].reshape(KK * MB, DIM).astype(bf16)
    qkv = jnp.dot(xall, wc_ref[...], preferred_element_type=f32)
    qkv = qkv + rxc * w2q + ryc * w2r
    f64 = rxc * ax + ryc * ay
    c64 = jnp.cos(f64)
    s64 = jnp.sin(f64)
    cT = jnp.concatenate([c64] * HEADS, axis=1)
    sT = jnp.concatenate([s64] * HEADS, axis=1)
    q = qkv[:, 0:INNER]
    qP = qkv[:, INNER:2 * INNER]
    kq = qkv[:, 2 * INNER:3 * INNER]
    kP = qkv[:, 3 * INNER:4 * INNER]
    qr = ((q * cT + qP * sT) * SCALE).astype(bf16)
    kr = (kq * cT + kP * sT).astype(bf16)
    krcat = jnp.concatenate(
        [kr[j * MB:(j + 1) * MB] for j in range(KK)], axis=1)
    aacc = jnp.zeros((MB, HEADS * KK), dtype=f32)
    for i in range(KK):
        qi = qr[i * MB:(i + 1) * MB]
        zi = jnp.concatenate([qi] * KK, axis=1) * krcat
        di = jnp.dot(zi, sbig_ref[...], preferred_element_type=f32)
        ei = jnp.exp(di)
        si = ei[:, 0:HEADS]
        for j in range(1, KK):
            si = si + ei[:, j * HEADS:(j + 1) * HEADS]
        ri = 1.0 / si
        rexp = jnp.concatenate([ri] * KK, axis=1)
        aacc = aacc + ei * rexp
    abar = (aacc * (1.0 / KK)).astype(bf16)
    pooled = jnp.zeros((MB, INNER), dtype=f32)
    for j in range(KK):
        aexp = jnp.dot(abar, ub_ref[j], preferred_element_type=f32)
        pooled = pooled + aexp * qkv[:, 4 * INNER:5 * INNER][j * MB:(j + 1) * MB]
    out = jnp.dot(pooled.astype(bf16), wout_ref[...], preferred_element_type=f32)
    out_ref[0, 0] = out + bout_ref[...]


def _dense(xg, rxc, ryc, wc, w2, axay, sbig, tmat, tmatT, ub, wout, bout):
    nj = M // MB
    return pl.pallas_call(
        _dense_body,
        grid=(B, T, nj),
        in_specs=[
            pl.BlockSpec((KK, MB, DIM),
                         lambda b, t, j: (0, (b * T + t) * nj + j, 0)),
            pl.BlockSpec((1, KK, MB, 1), lambda b, t, j: (b, 0, j, 0)),
            pl.BlockSpec((1, KK, MB, 1), lambda b, t, j: (b, 0, j, 0)),
            pl.BlockSpec((DIM, 5 * INNER), lambda b, t, j: (0, 0)),
            pl.BlockSpec((2, 5 * INNER), lambda b, t, j: (0, 0)),
            pl.BlockSpec((2, DIM_HEAD), lambda b, t, j: (0, 0)),
            pl.BlockSpec((KK * INNER, HEADS * KK), lambda b, t, j: (0, 0)),
            pl.BlockSpec((HEADS * KK, HEADS), lambda b, t, j: (0, 0)),
            pl.BlockSpec((HEADS, HEADS * KK), lambda b, t, j: (0, 0)),
            pl.BlockSpec((KK, HEADS * KK, INNER), lambda b, t, j: (0, 0, 0)),
            pl.BlockSpec((INNER, DIM), lambda b, t, j: (0, 0)),
            pl.BlockSpec((1, DIM), lambda b, t, j: (0, 0)),
        ],
        out_specs=pl.BlockSpec((1, 1, MB, DIM), lambda b, t, j: (b, t, j, 0)),
        out_shape=jax.ShapeDtypeStruct((B, T, M, DIM), f32),
    )(xg.reshape(KK, G, DIM), rxc, ryc, wc, w2, axay, sbig, tmat, tmatT,
      ub, wout, bout)


def _structure_mats():
    sbig = np.zeros((KK * INNER, HEADS * KK), dtype=np.float32)
    for j in range(KK):
        for h in range(HEADS):
            sbig[j * INNER + h * DIM_HEAD:(j * INNER + (h + 1) * DIM_HEAD),
                 j * HEADS + h] = 1.0
    tmat = np.zeros((HEADS * KK, HEADS), dtype=np.float32)
    for h in range(HEADS):
        tmat[h * KK:(h + 1) * KK, h] = 1.0
    ub = np.zeros((KK, HEADS * KK, INNER), dtype=np.float32)
    for j in range(KK):
        for h in range(HEADS):
            ub[j, j * HEADS + h, h * DIM_HEAD:(h + 1) * DIM_HEAD] = 1.0
    return sbig, tmat, tmat.T, ub


_SBIG_NP, _TMAT_NP, _TMATT_NP, _UB_NP = _structure_mats()


def _rot_cols(w):
    wh = w.reshape(-1, HEADS, 4, RD // 2)
    a, b2, c, d = wh[:, :, 0], wh[:, :, 1], wh[:, :, 2], wh[:, :, 3]
    return jnp.stack([-b2, a, -d, c], axis=2).reshape(-1, INNER)


def kernel(x, pos_embedding, W_qkv, W_out, b_out):
    px = pos_embedding[:, :, 0]
    py = pos_embedding[:, :, 1]
    pivx, pivy = _fps(px, py)
    nbr, relx, rely = _knn(px[..., None], py[..., None], pivx, pivy)

    bt_base = (jnp.arange(B * T, dtype=jnp.int32) * N).reshape(B, 1, T, 1)
    flat = (nbr[:, :, None, :] + bt_base)
    flat = jnp.transpose(flat, (1, 0, 2, 3)).reshape(ROWS)
    xg = _gather(x.reshape(B * T * N, DIM), flat)

    wq = W_qkv[:, 0:INNER]
    wk = W_qkv[:, INNER:2 * INNER]
    wv = W_qkv[:, 2 * INNER:3 * INNER]
    wcat = jnp.concatenate(
        [wq, _rot_cols(wq), wk, _rot_cols(wk), wv], axis=1)
    wc = wcat[:DIM].astype(bf16)
    w2 = wcat[DIM:DIM + 2]
    invf = 1.0 / (10000.0 ** (jnp.arange(0, RD, 2, dtype=f32) / RD))
    base = jnp.concatenate([invf, invf]) * 2048.0
    zeros = jnp.zeros((RD,), dtype=f32)
    axay = jnp.stack([jnp.concatenate([base, zeros]),
                      jnp.concatenate([zeros, base])])

    out = _dense(xg, relx[..., None], rely[..., None],
                 wc, w2, axay,
                 jnp.asarray(_SBIG_NP, dtype=bf16),
                 jnp.asarray(_TMAT_NP, dtype=bf16),
                 jnp.asarray(_TMATT_NP, dtype=bf16),
                 jnp.asarray(_UB_NP, dtype=bf16),
                 W_out.astype(bf16), b_out.reshape(1, DIM))
    pivot_pos = jnp.stack([pivx, pivy], axis=-1)
    return out, pivot_pos


# speedup vs baseline: 8.7511x; 1.0185x over previous
import functools

import jax
import jax.numpy as jnp
import numpy as np
from jax import lax
from jax.experimental import pallas as pl
from jax.experimental.pallas import tpu as pltpu
from jax.experimental.pallas import tpu_sc as plsc

B, T, N, DIM = 2, 4, 4096, 256
HEADS, DIM_HEAD, POOL = 8, 64, 8
M = N // POOL
KK = POOL + 1
INNER = HEADS * DIM_HEAD
RD = DIM_HEAD // 2
SCALE = DIM_HEAD ** -0.5
G = B * T * M
ROWS = KK * G
MB = 256

f32 = jnp.float32
bf16 = jnp.bfloat16


_FR = N // 128


def _fps_body(px_ref, py_ref, pivx_ref, pivy_ref):
    px = px_ref[...].reshape(B, _FR, 128)
    py = py_ref[...].reshape(B, _FR, 128)
    r32 = lax.broadcasted_iota(jnp.int32, (B, _FR, 128), 1)
    lane = lax.broadcasted_iota(jnp.int32, (B, _FR, 128), 2)
    gid = r32 * 128 + lane
    col = lax.broadcasted_iota(jnp.int32, (B, M), 1)
    x0 = px[:, 0:1, 0:1]
    y0 = py[:, 0:1, 0:1]
    dx = px - x0
    dy = py - y0
    mind0 = dx * dx + dy * dy
    pivx_ref[...] = jnp.where(col == 0, x0[:, 0], 0.0)
    pivy_ref[...] = jnp.where(col == 0, y0[:, 0], 0.0)

    def step(i, mind):
        am = jnp.max(jnp.max(mind, axis=1), axis=1)[:, None, None]
        idx = jnp.where(mind == am, gid, N)
        idx = jnp.min(jnp.min(idx, axis=1), axis=1)[:, None, None]
        sel = gid == idx
        nx = jnp.sum(jnp.sum(jnp.where(sel, px, 0.0), axis=1), axis=1)
        ny = jnp.sum(jnp.sum(jnp.where(sel, py, 0.0), axis=1), axis=1)
        nxb = nx[:, None, None]
        nyb = ny[:, None, None]
        ddx = px - nxb
        ddy = py - nyb
        nd = ddx * ddx + ddy * ddy
        pivx_ref[...] = jnp.where(col == i, nx[:, None], pivx_ref[...])
        pivy_ref[...] = jnp.where(col == i, ny[:, None], pivy_ref[...])
        return jnp.minimum(mind, nd)

    lax.fori_loop(1, M, step, mind0)


def _fps(px, py):
    return pl.pallas_call(
        _fps_body,
        out_shape=[jax.ShapeDtypeStruct((B, M), f32),
                   jax.ShapeDtypeStruct((B, M), f32)],
    )(px.reshape(B * _FR, 128), py.reshape(B * _FR, 128))


def _knn_body(pxc_ref, pyc_ref, pivx_ref, pivy_ref,
              nbr_ref, relx_ref, rely_ref, pd_ref):
    pxc = pxc_ref[0]
    pyc = pyc_ref[0]
    pivx = pivx_ref[0]
    pivy = pivy_ref[0]
    pxb = pxc.astype(bf16).astype(f32)
    pyb = pyc.astype(bf16).astype(f32)
    pvxb = pivx.astype(bf16).astype(f32)
    pvyb = pivy.astype(bf16).astype(f32)
    t = pvxb * pxb + pvyb * pyb
    inner = -2.0 * t
    xx = pivx * pivx + pivy * pivy
    yy = pxc * pxc + pyc * pyc
    pd_ref[...] = (-xx) - inner - yy
    rows = lax.broadcasted_iota(jnp.int32, (N, M), 0)
    for j in range(KK):
        pdv = pd_ref[...]
        amax = jnp.max(pdv, axis=0, keepdims=True)
        idx = jnp.min(jnp.where(pdv == amax, rows, N), axis=0, keepdims=True)
        sel = rows == idx
        gx = jnp.sum(jnp.where(sel, pxc, 0.0), axis=0, keepdims=True)
        gy = jnp.sum(jnp.where(sel, pyc, 0.0), axis=0, keepdims=True)
        nbr_ref[0, j] = idx[0]
        relx_ref[0, j] = (gx - pivx)[0]
        rely_ref[0, j] = (gy - pivy)[0]
        pd_ref[...] = jnp.where(sel, -jnp.inf, pdv)


def _knn(pxc, pyc, pivx, pivy):
    return pl.pallas_call(
        _knn_body,
        grid=(B,),
        in_specs=[
            pl.BlockSpec((1, N, 1), lambda b: (b, 0, 0)),
            pl.BlockSpec((1, N, 1), lambda b: (b, 0, 0)),
            pl.BlockSpec((1, 1, M), lambda b: (b, 0, 0)),
            pl.BlockSpec((1, 1, M), lambda b: (b, 0, 0)),
        ],
        out_specs=[
            pl.BlockSpec((1, KK, M), lambda b: (b, 0, 0)),
            pl.BlockSpec((1, KK, M), lambda b: (b, 0, 0)),
            pl.BlockSpec((1, KK, M), lambda b: (b, 0, 0)),
        ],
        out_shape=[jax.ShapeDtypeStruct((B, KK, M), jnp.int32),
                   jax.ShapeDtypeStruct((B, KK, M), f32),
                   jax.ShapeDtypeStruct((B, KK, M), f32)],
        scratch_shapes=[pltpu.VMEM((N, M), f32)],
    )(pxc, pyc, pivx.reshape(B, 1, M), pivy.reshape(B, 1, M))


_NWRK = 32
_ROWS_W = ROWS // _NWRK
_CH = 128
_NCH = _ROWS_W // _CH


def _gather_body(xf_hbm, idx_hbm, out_hbm, idx_v, rows_a, rows_b, sem_a, sem_b):
    c = lax.axis_index("c")
    s = lax.axis_index("s")
    wid = s * 2 + c
    base = wid * _ROWS_W
    pltpu.sync_copy(idx_hbm.at[pl.ds(base, _ROWS_W)], idx_v)
    bufs = (rows_a, rows_b)
    sems = (sem_a, sem_b)
    cps = [None, None]
    cps[0] = pltpu.async_copy(xf_hbm.at[idx_v.at[pl.ds(0, _CH)]], rows_a, sem_a)
    for ch in range(_NCH):
        nxt = ch + 1
        if nxt < _NCH:
            cps[nxt % 2] = pltpu.async_copy(
                xf_hbm.at[idx_v.at[pl.ds(nxt * _CH, _CH)]],
                bufs[nxt % 2], sems[nxt % 2])
        cps[ch % 2].wait()
        pltpu.sync_copy(bufs[ch % 2], out_hbm.at[pl.ds(base + ch * _CH, _CH)])


def _gather(xf, flat_idx):
    mesh = plsc.VectorSubcoreMesh(core_axis_name="c", subcore_axis_name="s")
    k = functools.partial(
        pl.kernel,
        mesh=mesh,
        out_type=jax.ShapeDtypeStruct((ROWS, DIM), f32),
        scratch_types=[
            pltpu.VMEM((_ROWS_W,), jnp.int32),
            pltpu.VMEM((_CH, DIM), f32),
            pltpu.VMEM((_CH, DIM), f32),
            pltpu.SemaphoreType.DMA,
            pltpu.SemaphoreType.DMA,
        ],
    )(_gather_body)
    return k(xf, flat_idx)


def _dense_body(xg_ref, rxc_ref, ryc_ref, wc_ref, w2_ref, axay_ref,
                sbig_ref, tmat_ref, tmatT_ref, ub_ref, wout_ref, bout_ref,
                out_ref):
    rxc = rxc_ref[0].reshape(KK * MB, 1)
    ryc = ryc_ref[0].reshape(KK * MB, 1)
    ax = axay_ref[0:1, :]
    ay = axay_ref[1:2, :]
    w2q = w2_ref[0:1, :]
    w2r = w2_ref[1:2, :]
    xall = xg_ref[...].reshape(KK * MB, DIM).astype(bf16)
    qkv = jnp.dot(xall, wc_ref[...], preferred_element_type=f32)
    qkv = qkv + rxc * w2q + ryc * w2r
    f64 = rxc * ax + ryc * ay
    c64 = jnp.cos(f64)
    s64 = jnp.sin(f64)
    cT = jnp.concatenate([c64] * HEADS, axis=1)
    sT = jnp.concatenate([s64] * HEADS, axis=1)
    q = qkv[:, 0:INNER]
    qP = qkv[:, INNER:2 * INNER]
    kq = qkv[:, 2 * INNER:3 * INNER]
    kP = qkv[:, 3 * INNER:4 * INNER]
    qr = ((q * cT + qP * sT) * SCALE).astype(bf16)
    kr = (kq * cT + kP * sT).astype(bf16)
    krcat = jnp.concatenate(
        [kr[j * MB:(j + 1) * MB] for j in range(KK)], axis=1)
    aacc = jnp.zeros((MB, HEADS * KK), dtype=f32)
    for i in range(KK):
        qi = qr[i * MB:(i + 1) * MB]
        zi = jnp.concatenate([qi] * KK, axis=1) * krcat
        di = jnp.dot(zi, sbig_ref[...], preferred_element_type=f32)
        ei = jnp.exp(di)
        si = ei[:, 0:HEADS]
        for j in range(1, KK):
            si = si + ei[:, j * HEADS:(j + 1) * HEADS]
        ri = 1.0 / si
        rexp = jnp.concatenate([ri] * KK, axis=1)
        aacc = aacc + ei * rexp
    abar = (aacc * (1.0 / KK)).astype(bf16)
    pooled = jnp.zeros((MB, INNER), dtype=f32)
    for j in range(KK):
        aexp = jnp.dot(abar, ub_ref[j], preferred_element_type=f32)
        pooled = pooled + aexp * qkv[:, 4 * INNER:5 * INNER][j * MB:(j + 1) * MB]
    out = jnp.dot(pooled.astype(bf16), wout_ref[...], preferred_element_type=f32)
    out_ref[0, 0] = out + bout_ref[...]


def _dense(xg, rxc, ryc, wc, w2, axay, sbig, tmat, tmatT, ub, wout, bout):
    nj = M // MB
    return pl.pallas_call(
        _dense_body,
        grid=(B, T, nj),
        in_specs=[
            pl.BlockSpec((KK, MB, DIM),
                         lambda b, t, j: (0, (b * T + t) * nj + j, 0)),
            pl.BlockSpec((1, KK, MB, 1), lambda b, t, j: (b, 0, j, 0)),
            pl.BlockSpec((1, KK, MB, 1), lambda b, t, j: (b, 0, j, 0)),
            pl.BlockSpec((DIM, 5 * INNER), lambda b, t, j: (0, 0)),
            pl.BlockSpec((2, 5 * INNER), lambda b, t, j: (0, 0)),
            pl.BlockSpec((2, DIM_HEAD), lambda b, t, j: (0, 0)),
            pl.BlockSpec((KK * INNER, HEADS * KK), lambda b, t, j: (0, 0)),
            pl.BlockSpec((HEADS * KK, HEADS), lambda b, t, j: (0, 0)),
            pl.BlockSpec((HEADS, HEADS * KK), lambda b, t, j: (0, 0)),
            pl.BlockSpec((KK, HEADS * KK, INNER), lambda b, t, j: (0, 0, 0)),
            pl.BlockSpec((INNER, DIM), lambda b, t, j: (0, 0)),
            pl.BlockSpec((1, DIM), lambda b, t, j: (0, 0)),
        ],
        out_specs=pl.BlockSpec((1, 1, MB, DIM), lambda b, t, j: (b, t, j, 0)),
        out_shape=jax.ShapeDtypeStruct((B, T, M, DIM), f32),
    )(xg.reshape(KK, G, DIM), rxc, ryc, wc, w2, axay, sbig, tmat, tmatT,
      ub, wout, bout)


def _structure_mats():
    sbig = np.zeros((KK * INNER, HEADS * KK), dtype=np.float32)
    for j in range(KK):
        for h in range(HEADS):
            sbig[j * INNER + h * DIM_HEAD:(j * INNER + (h + 1) * DIM_HEAD),
                 j * HEADS + h] = 1.0
    tmat = np.zeros((HEADS * KK, HEADS), dtype=np.float32)
    for h in range(HEADS):
        tmat[h * KK:(h + 1) * KK, h] = 1.0
    ub = np.zeros((KK, HEADS * KK, INNER), dtype=np.float32)
    for j in range(KK):
        for h in range(HEADS):
            ub[j, j * HEADS + h, h * DIM_HEAD:(h + 1) * DIM_HEAD] = 1.0
    return sbig, tmat, tmat.T, ub


_SBIG_NP, _TMAT_NP, _TMATT_NP, _UB_NP = _structure_mats()


def _rot_cols(w):
    wh = w.reshape(-1, HEADS, 4, RD // 2)
    a, b2, c, d = wh[:, :, 0], wh[:, :, 1], wh[:, :, 2], wh[:, :, 3]
    return jnp.stack([-b2, a, -d, c], axis=2).reshape(-1, INNER)


def kernel(x, pos_embedding, W_qkv, W_out, b_out):
    px = pos_embedding[:, :, 0]
    py = pos_embedding[:, :, 1]
    pivx, pivy = _fps(px, py)
    nbr, relx, rely = _knn(px[..., None], py[..., None], pivx, pivy)

    bt_base = (jnp.arange(B * T, dtype=jnp.int32) * N).reshape(B, 1, T, 1)
    flat = (nbr[:, :, None, :] + bt_base)
    flat = jnp.transpose(flat, (1, 0, 2, 3)).reshape(ROWS)
    xg = _gather(x.reshape(B * T * N, DIM), flat)

    wq = W_qkv[:, 0:INNER]
    wk = W_qkv[:, INNER:2 * INNER]
    wv = W_qkv[:, 2 * INNER:3 * INNER]
    wcat = jnp.concatenate(
        [wq, _rot_cols(wq), wk, _rot_cols(wk), wv], axis=1)
    wc = wcat[:DIM].astype(bf16)
    w2 = wcat[DIM:DIM + 2]
    invf = 1.0 / (10000.0 ** (jnp.arange(0, RD, 2, dtype=f32) / RD))
    base = jnp.concatenate([invf, invf]) * 2048.0
    zeros = jnp.zeros((RD,), dtype=f32)
    axay = jnp.stack([jnp.concatenate([base, zeros]),
                      jnp.concatenate([zeros, base])])

    out = _dense(xg, relx[..., None], rely[..., None],
                 wc, w2, axay,
                 jnp.asarray(_SBIG_NP, dtype=bf16),
                 jnp.asarray(_TMAT_NP, dtype=bf16),
                 jnp.asarray(_TMATT_NP, dtype=bf16),
                 jnp.asarray(_UB_NP, dtype=bf16),
                 W_out.astype(bf16), b_out.reshape(1, DIM))
    pivot_pos = jnp.stack([pivx, pivy], axis=-1)
    return out, pivot_pos
